# baseline (speedup 1.0000x reference)
import functools

import jax
import jax.numpy as jnp
from jax import lax
from jax.experimental import pallas as pl
from jax.experimental.pallas import tpu as pltpu
from jax.experimental.pallas import tpu_sc as plsc

N_NODES = 10000
N_EDGES = 320000
HID = 64
N_PAIR = N_EDGES // 2
_CROWS = 64
_NW = 32
_NCHUNKS = N_PAIR // _CROWS
_STRIPE = 624
_STRIPE_LAST = N_NODES - 15 * _STRIPE
_NITER = 79


def _dot(a, b):
    return jax.lax.dot_general(a, b, (((1,), (0,)), ((), ())),
                               precision=jax.lax.Precision.HIGHEST,
                               preferred_element_type=jnp.float32)


def _mlp4(x, ws):
    h = x
    for i, (W, b) in enumerate(ws):
        h = _dot(h, W) + b
        if i < 3:
            h = jnp.maximum(h, 0.0)
    return h


def _blockdiag_params(params):
    out = []
    for W, b in params:
        a, c = W.shape
        z = jnp.zeros((a, c), jnp.float32)
        Wbd = jnp.concatenate([
            jnp.concatenate([W, z], axis=1),
            jnp.concatenate([z, W], axis=1),
        ], axis=0)
        out.append((Wbd, jnp.concatenate([b, b])))
    return out


def _flatten_params(params):
    return [a for (W, b) in params for a in (W, b.reshape(1, -1))]


def _read_ws(refs):
    return [(refs[2 * i][...], refs[2 * i + 1][...]) for i in range(4)]


def _w_specs(flat):
    return [pl.BlockSpec(w.shape, lambda i: (0,) * w.ndim) for w in flat]


def _mlp_rows(x, params, tile, d_out=None):
    n, d_in = x.shape
    if d_out is None:
        d_out = params[3][0].shape[1]
    flat = _flatten_params(params)

    def body(x_ref, *refs):
        out_ref = refs[-1]
        out_ref[...] = _mlp4(x_ref[...], _read_ws(refs))

    return pl.pallas_call(
        body,
        grid=(pl.cdiv(n, tile),),
        in_specs=[pl.BlockSpec((tile, d_in), lambda i: (i, 0))] + _w_specs(flat),
        out_specs=pl.BlockSpec((tile, d_out), lambda i: (i, 0)),
        out_shape=jax.ShapeDtypeStruct((n, d_out), jnp.float32),
    )(x, *flat)


def _sr_mlps(node_h, sparams, rparams, tile):
    n, d = node_h.shape
    sflat = _flatten_params(sparams)
    rflat = _flatten_params(rparams)

    def body(h_ref, *refs):
        s_out, r_out = refs[-2], refs[-1]
        h = h_ref[...]
        s_out[...] = _mlp4(h, _read_ws(refs[0:8]))
        r_out[...] = _mlp4(h, _read_ws(refs[8:16]))

    return pl.pallas_call(
        body,
        grid=(pl.cdiv(n, tile),),
        in_specs=[pl.BlockSpec((tile, d), lambda i: (i, 0))]
        + _w_specs(sflat) + _w_specs(rflat),
        out_specs=[pl.BlockSpec((tile, HID), lambda i: (i, 0))] * 2,
        out_shape=[jax.ShapeDtypeStruct((n, HID), jnp.float32)] * 2,
    )(node_h, *sflat, *rflat)


def _fused_add_mlp(msg2, res2, eparams_bd, tile):
    n, d = msg2.shape
    flat = _flatten_params(eparams_bd)

    def body(m_ref, r_ref, *refs):
        e_ref, h_ref = refs[-2], refs[-1]
        h = m_ref[...] + r_ref[...]
        h_ref[...] = h
        e_ref[...] = _mlp4(h, _read_ws(refs))

    return pl.pallas_call(
        body,
        grid=(pl.cdiv(n, tile),),
        in_specs=[pl.BlockSpec((tile, d), lambda i: (i, 0))] * 2 + _w_specs(flat),
        out_specs=[pl.BlockSpec((tile, 2 * HID), lambda i: (i, 0))] * 2,
        out_shape=[jax.ShapeDtypeStruct((n, 2 * HID), jnp.float32)] * 2,
    )(msg2, res2, *flat)


def _node_block2(node_h, agg_parts, nparams, tile):
    n, d = node_h.shape
    (W1, b1), (W2, b2), (W3, b3), (W4, b4) = nparams
    W1a, W1b = W1[:d], W1[d:]
    flat = [W1a, W1b, b1.reshape(1, -1), W2, b2.reshape(1, -1),
            W3, b3.reshape(1, -1), W4, b4.reshape(1, -1)]

    def body(h_ref, a_ref, w1a, w1b, b1r, w2, b2r, w3, b3r, w4, b4r, out_ref):
        h = h_ref[...]
        a = a_ref[0] + a_ref[1]
        z = jnp.maximum(_dot(h, w1a[...]) + _dot(a, w1b[...]) + b1r[...], 0.0)
        z = jnp.maximum(_dot(z, w2[...]) + b2r[...], 0.0)
        z = jnp.maximum(_dot(z, w3[...]) + b3r[...], 0.0)
        out_ref[...] = _dot(z, w4[...]) + b4r[...] + h

    return pl.pallas_call(
        body,
        grid=(pl.cdiv(n, tile),),
        in_specs=[pl.BlockSpec((tile, d), lambda i: (i, 0)),
                  pl.BlockSpec((2, tile, HID), lambda i: (0, i, 0))]
        + _w_specs(flat),
        out_specs=pl.BlockSpec((tile, HID), lambda i: (i, 0)),
        out_shape=jax.ShapeDtypeStruct((n, HID), jnp.float32),
    )(node_h, agg_parts, *flat)


def _sc_edge_pass(E2, idx_s, idx_r, S, R):
    mesh = plsc.VectorSubcoreMesh(core_axis_name="c", subcore_axis_name="s")

    @functools.partial(
        pl.kernel,
        mesh=mesh,
        compiler_params=pltpu.CompilerParams(use_tc_tiling_on_sc=False),
        out_type=[
            jax.ShapeDtypeStruct((N_PAIR, 2 * HID), jnp.float32),
            jax.ShapeDtypeStruct((2, N_NODES, HID), jnp.float32),
        ],
        scratch_types=[
            [pltpu.VMEM((2 * _CROWS,), jnp.int32) for _ in range(6)],
            [pltpu.VMEM((2 * _CROWS,), jnp.int32) for _ in range(6)],
            [pltpu.VMEM((_CROWS, 2 * HID), jnp.float32) for _ in range(3)],
            [pltpu.VMEM((2 * _CROWS, HID), jnp.float32) for _ in range(3)],
            [pltpu.VMEM((2 * _CROWS, HID), jnp.float32) for _ in range(3)],
            pltpu.VMEM((160, HID), jnp.float32),
            pltpu.VMEM_SHARED((N_NODES, HID), jnp.float32),
            [pltpu.SemaphoreType.DMA for _ in range(3)],
            [pltpu.SemaphoreType.DMA for _ in range(3)],
            pltpu.SemaphoreType.DMA,
        ],
    )
    def body(e_hbm, si_hbm, ri_hbm, sv_hbm, rv_hbm, msg_hbm, agg_hbm,
             idxs6, idxr6, ebuf, sbuf, rbuf, stage, agg_sh,
             lsem, gsem, isem):
        cid = lax.axis_index("c")
        sid = lax.axis_index("s")
        wid = sid * 2 + cid

        def zero_stage(i, _):
            for q in range(HID // 16):
                stage[i, pl.ds(q * 16, 16)] = jnp.zeros((16,), jnp.float32)
            return 0
        lax.fori_loop(0, 160, zero_stage, 0)
        sbase = sid * _STRIPE

        @pl.when(sid < 15)
        def _():
            for off, nr in ((0, 160), (160, 160), (320, 160), (480, 144)):
                pltpu.sync_copy(stage.at[pl.ds(0, nr)],
                                agg_sh.at[pl.ds(sbase + off, nr)])

        @pl.when(sid == 15)
        def _():
            for off in (0, 160, 320, 480):
                pltpu.sync_copy(stage, agg_sh.at[pl.ds(15 * _STRIPE + off, 160)])
        plsc.subcore_barrier()

        def idx_sync(j, bi):
            t = wid + _NW * j
            pltpu.sync_copy(si_hbm.at[t], idxs6[bi])
            pltpu.sync_copy(ri_hbm.at[t], idxr6[bi])

        def idx_async(j, bi):
            t = wid + _NW * j
            pltpu.async_copy(si_hbm.at[t], idxs6[bi], isem)
            pltpu.async_copy(ri_hbm.at[t], idxr6[bi], isem)

        def drain_idx():
            pltpu.make_async_copy(si_hbm.at[0], idxs6[0], isem).wait()
            pltpu.make_async_copy(ri_hbm.at[0], idxr6[0], isem).wait()

        def issue_inputs(j, b, bi):
            t = wid + _NW * j
            base = t * _CROWS
            pltpu.async_copy(e_hbm.at[pl.ds(base, _CROWS)], ebuf[b], lsem[b])
            pltpu.async_copy(sv_hbm.at[idxs6[bi]], sbuf[b], gsem[b])
            pltpu.async_copy(rv_hbm.at[idxr6[bi]], rbuf[b], gsem[b])

        def drain(sem, n):
            for _ in range(n):
                pltpu.make_async_copy(e_hbm.at[pl.ds(0, _CROWS)],
                                      ebuf[0], sem).wait()

        idx_sync(0, 0)
        idx_sync(1, 1)
        idx_async(2, 2)
        issue_inputs(0, 0, 0)
        issue_inputs(1, 1, 1)

        def outer(g, _):
            for u in range(6):
                b = u % 3
                j = 6 * g + u
                t = wid + _NW * j

                @pl.when(t < _NCHUNKS)
                def _():
                    drain(lsem[b], 1)
                    drain(gsem[b], 2)

                    def addrow(i, _):
                        for q in range(HID // 16):
                            sl = pl.ds(q * 16, 16)
                            sh = pl.ds(HID + q * 16, 16)
                            ve = ebuf[b][i, sl] + sbuf[b][i, sl] + rbuf[b][i, sl]
                            vo = (ebuf[b][i, sh] + sbuf[b][_CROWS + i, sl]
                                  + rbuf[b][_CROWS + i, sl])
                            ebuf[b][i, sl] = ve
                            ebuf[b][i, sh] = vo
                            sbuf[b][i, sl] = ve
                            sbuf[b][_CROWS + i, sl] = vo
                        return 0
                    lax.fori_loop(0, _CROWS, addrow, 0)

                    pltpu.sync_copy(sbuf[b], agg_sh.at[idxr6[u]], add=True)
                    pltpu.sync_copy(ebuf[b],
                                    msg_hbm.at[pl.ds(t * _CROWS, _CROWS)])

                b2 = (b + 2) % 3
                u2 = (u + 2) % 6
                t2 = wid + _NW * (j + 2)
                t3 = wid + _NW * (j + 3)

                @pl.when(t2 < _NCHUNKS)
                def _():
                    drain_idx()
                    issue_inputs(j + 2, b2, u2)

                @pl.when(t3 < _NCHUNKS)
                def _():
                    idx_async(j + 3, (u + 3) % 6)
            return 0
        lax.fori_loop(0, (_NITER + 5) // 6, outer, 0)

        plsc.subcore_barrier()

        @pl.when(sid < 15)
        def _():
            for off, nr in ((0, 160), (160, 160), (320, 160), (480, 144)):
                pltpu.sync_copy(agg_sh.at[pl.ds(sbase + off, nr)],
                                stage.at[pl.ds(0, nr)])
                pltpu.sync_copy(stage.at[pl.ds(0, nr)],
                                agg_hbm.at[cid, pl.ds(sbase + off, nr)])

        @pl.when(sid == 15)
        def _():
            for off in (0, 160, 320, 480):
                pltpu.sync_copy(agg_sh.at[pl.ds(15 * _STRIPE + off, 160)], stage)
                pltpu.sync_copy(stage,
                                agg_hbm.at[cid, pl.ds(15 * _STRIPE + off, 160)])

    return body(E2, idx_s, idx_r, S, R)


def kernel(x, edge_attr, edge_index, params):
    senders = edge_index[0]
    receivers = edge_index[1]

    def _group_idx(v):
        c = v.reshape(_NCHUNKS, _CROWS, 2)
        c = jnp.concatenate([c[:, :, 0], c[:, :, 1]], axis=1)
        return jnp.pad(c, ((0, _NITER * _NW - _NCHUNKS), (0, 0)))

    idx_s = _group_idx(senders)
    idx_r = _group_idx(receivers)

    node_h = _mlp_rows(x, params['nb_encoder'], 2000)
    ea2 = edge_attr.reshape(N_PAIR, 32)
    h_prev = _mlp_rows(ea2, _blockdiag_params(params['eb_encoder']), 4000,
                       d_out=2 * HID)

    msg2 = None
    for k, blk in enumerate(params['blocks']):
        S, R = _sr_mlps(node_h, blk['sender'], blk['receiver'], 2000)
        ebd = _blockdiag_params(blk['edge'])
        if k == 0:
            E2 = _mlp_rows(h_prev, ebd, 4000, d_out=2 * HID)
        else:
            E2, h_prev = _fused_add_mlp(msg2, h_prev, ebd, 4000)
        msg2, agg_parts = _sc_edge_pass(E2, idx_s, idx_r, S, R)
        node_h = _node_block2(node_h, agg_parts, blk['node'], 2000)

    return _mlp_rows(node_h, params['decoder'], 2000)


# speedup vs baseline: 1.9907x; 1.9907x over previous
import functools

import jax
import jax.numpy as jnp
from jax import lax
from jax.experimental import pallas as pl
from jax.experimental.pallas import tpu as pltpu
from jax.experimental.pallas import tpu_sc as plsc

N_NODES = 10000
N_EDGES = 320000
HID = 64
N_PAIR = N_EDGES // 2
_CROWS = 64
_NW = 32
_NCHUNKS = N_PAIR // _CROWS
_STRIPE = 624
_STRIPE_LAST = N_NODES - 15 * _STRIPE
_NITER = 79


def _dot(a, b):
    return jax.lax.dot_general(a, b, (((1,), (0,)), ((), ())),
                               preferred_element_type=jnp.float32)


def _mlp4(x, ws):
    h = x
    for i, (W, b) in enumerate(ws):
        h = _dot(h, W) + b
        if i < 3:
            h = jnp.maximum(h, 0.0)
    return h


def _blockdiag_params(params):
    out = []
    for W, b in params:
        a, c = W.shape
        z = jnp.zeros((a, c), jnp.float32)
        Wbd = jnp.concatenate([
            jnp.concatenate([W, z], axis=1),
            jnp.concatenate([z, W], axis=1),
        ], axis=0)
        out.append((Wbd, jnp.concatenate([b, b])))
    return out


def _flatten_params(params):
    return [a for (W, b) in params for a in (W, b.reshape(1, -1))]


def _read_ws(refs):
    return [(refs[2 * i][...], refs[2 * i + 1][...]) for i in range(4)]


def _w_specs(flat):
    return [pl.BlockSpec(w.shape, lambda i: (0,) * w.ndim) for w in flat]


def _mlp_rows(x, params, tile, d_out=None):
    n, d_in = x.shape
    if d_out is None:
        d_out = params[3][0].shape[1]
    flat = _flatten_params(params)

    def body(x_ref, *refs):
        out_ref = refs[-1]
        out_ref[...] = _mlp4(x_ref[...], _read_ws(refs))

    return pl.pallas_call(
        body,
        grid=(pl.cdiv(n, tile),),
        in_specs=[pl.BlockSpec((tile, d_in), lambda i: (i, 0))] + _w_specs(flat),
        out_specs=pl.BlockSpec((tile, d_out), lambda i: (i, 0)),
        out_shape=jax.ShapeDtypeStruct((n, d_out), jnp.float32),
    )(x, *flat)


def _sr_mlps(node_h, sparams, rparams, tile):
    n, d = node_h.shape
    sflat = _flatten_params(sparams)
    rflat = _flatten_params(rparams)

    def body(h_ref, *refs):
        s_out, r_out = refs[-2], refs[-1]
        h = h_ref[...]
        s_out[...] = _mlp4(h, _read_ws(refs[0:8]))
        r_out[...] = _mlp4(h, _read_ws(refs[8:16]))

    return pl.pallas_call(
        body,
        grid=(pl.cdiv(n, tile),),
        in_specs=[pl.BlockSpec((tile, d), lambda i: (i, 0))]
        + _w_specs(sflat) + _w_specs(rflat),
        out_specs=[pl.BlockSpec((tile, HID), lambda i: (i, 0))] * 2,
        out_shape=[jax.ShapeDtypeStruct((n, HID), jnp.float32)] * 2,
    )(node_h, *sflat, *rflat)


def _fused_add_mlp(msg2, res2, eparams_bd, tile):
    n, d = msg2.shape
    flat = _flatten_params(eparams_bd)

    def body(m_ref, r_ref, *refs):
        e_ref, h_ref = refs[-2], refs[-1]
        h = m_ref[...] + r_ref[...]
        h_ref[...] = h
        e_ref[...] = _mlp4(h, _read_ws(refs))

    return pl.pallas_call(
        body,
        grid=(pl.cdiv(n, tile),),
        in_specs=[pl.BlockSpec((tile, d), lambda i: (i, 0))] * 2 + _w_specs(flat),
        out_specs=[pl.BlockSpec((tile, 2 * HID), lambda i: (i, 0))] * 2,
        out_shape=[jax.ShapeDtypeStruct((n, 2 * HID), jnp.float32)] * 2,
    )(msg2, res2, *flat)


def _node_block2(node_h, agg_parts, nparams, tile):
    n, d = node_h.shape
    flat = _flatten_params(nparams)

    def body(h_ref, a_ref, *refs):
        out_ref = refs[-1]
        h = h_ref[...]
        a = a_ref[0] + a_ref[1]
        z = jnp.concatenate([h, a], axis=1)
        out_ref[...] = _mlp4(z, _read_ws(refs)) + h

    return pl.pallas_call(
        body,
        grid=(pl.cdiv(n, tile),),
        in_specs=[pl.BlockSpec((tile, d), lambda i: (i, 0)),
                  pl.BlockSpec((2, tile, HID), lambda i: (0, i, 0))]
        + _w_specs(flat),
        out_specs=pl.BlockSpec((tile, HID), lambda i: (i, 0)),
        out_shape=jax.ShapeDtypeStruct((n, HID), jnp.float32),
    )(node_h, agg_parts, *flat)


def _sc_edge_pass(E2, idx_s, idx_r, S, R):
    mesh = plsc.VectorSubcoreMesh(core_axis_name="c", subcore_axis_name="s")

    @functools.partial(
        pl.kernel,
        mesh=mesh,
        compiler_params=pltpu.CompilerParams(use_tc_tiling_on_sc=False),
        out_type=[
            jax.ShapeDtypeStruct((N_PAIR, 2 * HID), jnp.float32),
            jax.ShapeDtypeStruct((2, N_NODES, HID), jnp.float32),
        ],
        scratch_types=[
            [pltpu.VMEM((2 * _CROWS,), jnp.int32) for _ in range(6)],
            [pltpu.VMEM((2 * _CROWS,), jnp.int32) for _ in range(6)],
            [pltpu.VMEM((_CROWS, 2 * HID), jnp.float32) for _ in range(3)],
            [pltpu.VMEM((2 * _CROWS, HID), jnp.float32) for _ in range(3)],
            [pltpu.VMEM((2 * _CROWS, HID), jnp.float32) for _ in range(3)],
            pltpu.VMEM((160, HID), jnp.float32),
            pltpu.VMEM_SHARED((N_NODES, HID), jnp.float32),
            [pltpu.SemaphoreType.DMA for _ in range(3)],
            [pltpu.SemaphoreType.DMA for _ in range(3)],
            pltpu.SemaphoreType.DMA,
        ],
    )
    def body(e_hbm, si_hbm, ri_hbm, sv_hbm, rv_hbm, msg_hbm, agg_hbm,
             idxs6, idxr6, ebuf, sbuf, rbuf, stage, agg_sh,
             lsem, gsem, isem):
        cid = lax.axis_index("c")
        sid = lax.axis_index("s")
        wid = sid * 2 + cid

        def zero_stage(i, _):
            for q in range(HID // 16):
                stage[i, pl.ds(q * 16, 16)] = jnp.zeros((16,), jnp.float32)
            return 0
        lax.fori_loop(0, 160, zero_stage, 0)
        sbase = sid * _STRIPE

        @pl.when(sid < 15)
        def _():
            for off, nr in ((0, 160), (160, 160), (320, 160), (480, 144)):
                pltpu.sync_copy(stage.at[pl.ds(0, nr)],
                                agg_sh.at[pl.ds(sbase + off, nr)])

        @pl.when(sid == 15)
        def _():
            for off in (0, 160, 320, 480):
                pltpu.sync_copy(stage, agg_sh.at[pl.ds(15 * _STRIPE + off, 160)])
        plsc.subcore_barrier()

        def idx_sync(j, bi):
            t = wid + _NW * j
            pltpu.sync_copy(si_hbm.at[t], idxs6[bi])
            pltpu.sync_copy(ri_hbm.at[t], idxr6[bi])

        def idx_async(j, bi):
            t = wid + _NW * j
            pltpu.async_copy(si_hbm.at[t], idxs6[bi], isem)
            pltpu.async_copy(ri_hbm.at[t], idxr6[bi], isem)

        def drain_idx():
            pltpu.make_async_copy(si_hbm.at[0], idxs6[0], isem).wait()
            pltpu.make_async_copy(ri_hbm.at[0], idxr6[0], isem).wait()

        def issue_inputs(j, b, bi):
            t = wid + _NW * j
            base = t * _CROWS
            pltpu.async_copy(e_hbm.at[pl.ds(base, _CROWS)], ebuf[b], lsem[b])
            pltpu.async_copy(sv_hbm.at[idxs6[bi]], sbuf[b], gsem[b])
            pltpu.async_copy(rv_hbm.at[idxr6[bi]], rbuf[b], gsem[b])

        def drain(sem, n):
            for _ in range(n):
                pltpu.make_async_copy(e_hbm.at[pl.ds(0, _CROWS)],
                                      ebuf[0], sem).wait()

        idx_sync(0, 0)
        idx_sync(1, 1)
        idx_async(2, 2)
        issue_inputs(0, 0, 0)
        issue_inputs(1, 1, 1)

        def outer(g, _):
            for u in range(6):
                b = u % 3
                j = 6 * g + u
                t = wid + _NW * j

                @pl.when(t < _NCHUNKS)
                def _():
                    drain(lsem[b], 1)
                    drain(gsem[b], 2)

                    def addrow(i, _):
                        for q in range(HID // 16):
                            sl = pl.ds(q * 16, 16)
                            sh = pl.ds(HID + q * 16, 16)
                            ve = ebuf[b][i, sl] + sbuf[b][i, sl] + rbuf[b][i, sl]
                            vo = (ebuf[b][i, sh] + sbuf[b][_CROWS + i, sl]
                                  + rbuf[b][_CROWS + i, sl])
                            ebuf[b][i, sl] = ve
                            ebuf[b][i, sh] = vo
                            sbuf[b][i, sl] = ve
                            sbuf[b][_CROWS + i, sl] = vo
                        return 0
                    lax.fori_loop(0, _CROWS, addrow, 0)

                    pltpu.sync_copy(sbuf[b], agg_sh.at[idxr6[u]], add=True)
                    pltpu.sync_copy(ebuf[b],
                                    msg_hbm.at[pl.ds(t * _CROWS, _CROWS)])

                b2 = (b + 2) % 3
                u2 = (u + 2) % 6
                t2 = wid + _NW * (j + 2)
                t3 = wid + _NW * (j + 3)

                @pl.when(t2 < _NCHUNKS)
                def _():
                    drain_idx()
                    issue_inputs(j + 2, b2, u2)

                @pl.when(t3 < _NCHUNKS)
                def _():
                    idx_async(j + 3, (u + 3) % 6)
            return 0
        lax.fori_loop(0, (_NITER + 5) // 6, outer, 0)

        plsc.subcore_barrier()

        @pl.when(sid < 15)
        def _():
            for off, nr in ((0, 160), (160, 160), (320, 160), (480, 144)):
                pltpu.sync_copy(agg_sh.at[pl.ds(sbase + off, nr)],
                                stage.at[pl.ds(0, nr)])
                pltpu.sync_copy(stage.at[pl.ds(0, nr)],
                                agg_hbm.at[cid, pl.ds(sbase + off, nr)])

        @pl.when(sid == 15)
        def _():
            for off in (0, 160, 320, 480):
                pltpu.sync_copy(agg_sh.at[pl.ds(15 * _STRIPE + off, 160)], stage)
                pltpu.sync_copy(stage,
                                agg_hbm.at[cid, pl.ds(15 * _STRIPE + off, 160)])

    return body(E2, idx_s, idx_r, S, R)


def kernel(x, edge_attr, edge_index, params):
    senders = edge_index[0]
    receivers = edge_index[1]

    def _group_idx(v):
        c = v.reshape(_NCHUNKS, _CROWS, 2)
        c = jnp.concatenate([c[:, :, 0], c[:, :, 1]], axis=1)
        return jnp.pad(c, ((0, _NITER * _NW - _NCHUNKS), (0, 0)))

    idx_s = _group_idx(senders)
    idx_r = _group_idx(receivers)

    node_h = _mlp_rows(x, params['nb_encoder'], 2000)
    ea2 = edge_attr.reshape(N_PAIR, 32)
    h_prev = _mlp_rows(ea2, _blockdiag_params(params['eb_encoder']), 4000,
                       d_out=2 * HID)

    msg2 = None
    for k, blk in enumerate(params['blocks']):
        S, R = _sr_mlps(node_h, blk['sender'], blk['receiver'], 2000)
        ebd = _blockdiag_params(blk['edge'])
        if k == 0:
            E2 = _mlp_rows(h_prev, ebd, 4000, d_out=2 * HID)
        else:
            E2, h_prev = _fused_add_mlp(msg2, h_prev, ebd, 4000)
        msg2, agg_parts = _sc_edge_pass(E2, idx_s, idx_r, S, R)
        node_h = _node_block2(node_h, agg_parts, blk['node'], 2000)

    return _mlp_rows(node_h, params['decoder'], 2000)


# speedup vs baseline: 3.1256x; 1.5701x over previous
import functools

import jax
import jax.numpy as jnp
from jax import lax
from jax.experimental import pallas as pl
from jax.experimental.pallas import tpu as pltpu
from jax.experimental.pallas import tpu_sc as plsc

N_NODES = 10000
N_EDGES = 320000
HID = 64
N_PAIR = N_EDGES // 2
_CROWS = 64
_NW = 32
_NCHUNKS = N_PAIR // _CROWS
_STRIPE = 624
_STRIPE_LAST = N_NODES - 15 * _STRIPE
_NITER = 79


def _dot(a, b):
    return jax.lax.dot_general(a, b, (((1,), (0,)), ((), ())),
                               preferred_element_type=jnp.float32)


def _mlp4(x, ws):
    h = x
    for i, (W, b) in enumerate(ws):
        h = _dot(h, W) + b
        if i < 3:
            h = jnp.maximum(h, 0.0)
    return h


def _blockdiag_params(params):
    out = []
    for W, b in params:
        a, c = W.shape
        z = jnp.zeros((a, c), jnp.float32)
        Wbd = jnp.concatenate([
            jnp.concatenate([W, z], axis=1),
            jnp.concatenate([z, W], axis=1),
        ], axis=0)
        out.append((Wbd, jnp.concatenate([b, b])))
    return out


def _flatten_params(params):
    return [a for (W, b) in params for a in (W, b.reshape(1, -1))]


def _read_ws(refs):
    return [(refs[2 * i][...], refs[2 * i + 1][...]) for i in range(4)]


def _w_specs(flat):
    return [pl.BlockSpec(w.shape, lambda i: (0,) * w.ndim) for w in flat]


def _mlp_rows(x, params, tile, d_out=None):
    n, d_in = x.shape
    if d_out is None:
        d_out = params[3][0].shape[1]
    flat = _flatten_params(params)

    def body(x_ref, *refs):
        out_ref = refs[-1]
        out_ref[...] = _mlp4(x_ref[...], _read_ws(refs))

    return pl.pallas_call(
        body,
        grid=(pl.cdiv(n, tile),),
        in_specs=[pl.BlockSpec((tile, d_in), lambda i: (i, 0))] + _w_specs(flat),
        out_specs=pl.BlockSpec((tile, d_out), lambda i: (i, 0)),
        out_shape=jax.ShapeDtypeStruct((n, d_out), jnp.float32),
    )(x, *flat)


def _sr_mlps(node_h, sparams, rparams, tile):
    n, d = node_h.shape
    sflat = _flatten_params(sparams)
    rflat = _flatten_params(rparams)

    def body(h_ref, *refs):
        s_out, r_out = refs[-2], refs[-1]
        h = h_ref[...]
        s_out[...] = _mlp4(h, _read_ws(refs[0:8]))
        r_out[...] = _mlp4(h, _read_ws(refs[8:16]))

    return pl.pallas_call(
        body,
        grid=(pl.cdiv(n, tile),),
        in_specs=[pl.BlockSpec((tile, d), lambda i: (i, 0))]
        + _w_specs(sflat) + _w_specs(rflat),
        out_specs=[pl.BlockSpec((tile, HID), lambda i: (i, 0))] * 2,
        out_shape=[jax.ShapeDtypeStruct((n, HID), jnp.float32)] * 2,
    )(node_h, *sflat, *rflat)


def _fused_add_mlp(msg2, res2, eparams_bd, tile):
    n, d = msg2.shape
    flat = _flatten_params(eparams_bd)

    def body(m_ref, r_ref, *refs):
        e_ref, h_ref = refs[-2], refs[-1]
        h = m_ref[...] + r_ref[...]
        h_ref[...] = h
        e_ref[...] = _mlp4(h, _read_ws(refs))

    return pl.pallas_call(
        body,
        grid=(pl.cdiv(n, tile),),
        in_specs=[pl.BlockSpec((tile, d), lambda i: (i, 0))] * 2 + _w_specs(flat),
        out_specs=[pl.BlockSpec((tile, 2 * HID), lambda i: (i, 0))] * 2,
        out_shape=[jax.ShapeDtypeStruct((n, 2 * HID), jnp.float32)] * 2,
    )(msg2, res2, *flat)


def _node_block2(node_h, agg_parts, nparams, tile):
    n, d = node_h.shape
    flat = _flatten_params(nparams)

    def body(h_ref, a_ref, *refs):
        out_ref = refs[-1]
        h = h_ref[...]
        a = a_ref[0] + a_ref[1]
        z = jnp.concatenate([h, a], axis=1)
        out_ref[...] = _mlp4(z, _read_ws(refs)) + h

    return pl.pallas_call(
        body,
        grid=(pl.cdiv(n, tile),),
        in_specs=[pl.BlockSpec((tile, d), lambda i: (i, 0)),
                  pl.BlockSpec((2, tile, HID), lambda i: (0, i, 0))]
        + _w_specs(flat),
        out_specs=pl.BlockSpec((tile, HID), lambda i: (i, 0)),
        out_shape=jax.ShapeDtypeStruct((n, HID), jnp.float32),
    )(node_h, agg_parts, *flat)


def _sc_edge_pass(E2, edge_index, S, R):
    mesh = plsc.VectorSubcoreMesh(core_axis_name="c", subcore_axis_name="s")

    @functools.partial(
        pl.kernel,
        mesh=mesh,
        compiler_params=pltpu.CompilerParams(use_tc_tiling_on_sc=False),
        out_type=[
            jax.ShapeDtypeStruct((N_PAIR, 2 * HID), jnp.float32),
            jax.ShapeDtypeStruct((2, N_NODES, HID), jnp.float32),
        ],
        scratch_types=[
            [pltpu.VMEM((2 * _CROWS,), jnp.int32) for _ in range(6)],
            [pltpu.VMEM((2 * _CROWS,), jnp.int32) for _ in range(6)],
            [pltpu.VMEM((_CROWS, 2 * HID), jnp.float32) for _ in range(3)],
            [pltpu.VMEM((2 * _CROWS, HID), jnp.float32) for _ in range(3)],
            [pltpu.VMEM((2 * _CROWS, HID), jnp.float32) for _ in range(3)],
            pltpu.VMEM((160, HID), jnp.float32),
            pltpu.VMEM_SHARED((N_NODES, HID), jnp.float32),
            [pltpu.SemaphoreType.DMA for _ in range(3)],
            [pltpu.SemaphoreType.DMA for _ in range(3)],
            pltpu.SemaphoreType.DMA,
        ],
    )
    def body(e_hbm, ei_hbm, sv_hbm, rv_hbm, msg_hbm, agg_hbm,
             idxs6, idxr6, ebuf, sbuf, rbuf, stage, agg_sh,
             lsem, gsem, isem):
        cid = lax.axis_index("c")
        sid = lax.axis_index("s")
        wid = sid * 2 + cid

        def zero_stage(i, _):
            for q in range(HID // 16):
                stage[i, pl.ds(q * 16, 16)] = jnp.zeros((16,), jnp.float32)
            return 0
        lax.fori_loop(0, 160, zero_stage, 0)
        sbase = sid * _STRIPE

        @pl.when(sid < 15)
        def _():
            for off, nr in ((0, 160), (160, 160), (320, 160), (480, 144)):
                pltpu.sync_copy(stage.at[pl.ds(0, nr)],
                                agg_sh.at[pl.ds(sbase + off, nr)])

        @pl.when(sid == 15)
        def _():
            for off in (0, 160, 320, 480):
                pltpu.sync_copy(stage, agg_sh.at[pl.ds(15 * _STRIPE + off, 160)])
        plsc.subcore_barrier()

        def idx_sync(j, bi):
            base = (wid + _NW * j) * 2 * _CROWS
            pltpu.sync_copy(ei_hbm.at[0, pl.ds(base, 2 * _CROWS)], idxs6[bi])
            pltpu.sync_copy(ei_hbm.at[1, pl.ds(base, 2 * _CROWS)], idxr6[bi])

        def idx_async(j, bi):
            base = (wid + _NW * j) * 2 * _CROWS
            pltpu.async_copy(ei_hbm.at[0, pl.ds(base, 2 * _CROWS)], idxs6[bi], isem)
            pltpu.async_copy(ei_hbm.at[1, pl.ds(base, 2 * _CROWS)], idxr6[bi], isem)

        def drain_idx():
            pltpu.make_async_copy(ei_hbm.at[0, pl.ds(0, 2 * _CROWS)],
                                  idxs6[0], isem).wait()
            pltpu.make_async_copy(ei_hbm.at[1, pl.ds(0, 2 * _CROWS)],
                                  idxr6[0], isem).wait()

        def issue_inputs(j, b, bi):
            t = wid + _NW * j
            base = t * _CROWS
            pltpu.async_copy(e_hbm.at[pl.ds(base, _CROWS)], ebuf[b], lsem[b])
            pltpu.async_copy(sv_hbm.at[idxs6[bi]], sbuf[b], gsem[b])
            pltpu.async_copy(rv_hbm.at[idxr6[bi]], rbuf[b], gsem[b])

        def drain(sem, n):
            for _ in range(n):
                pltpu.make_async_copy(e_hbm.at[pl.ds(0, _CROWS)],
                                      ebuf[0], sem).wait()

        idx_sync(0, 0)
        idx_sync(1, 1)
        idx_async(2, 2)
        issue_inputs(0, 0, 0)
        issue_inputs(1, 1, 1)

        def outer(g, _):
            for u in range(6):
                b = u % 3
                j = 6 * g + u
                t = wid + _NW * j

                @pl.when(t < _NCHUNKS)
                def _():
                    drain(lsem[b], 1)
                    drain(gsem[b], 2)

                    def addrow(i, _):
                        for q in range(HID // 16):
                            sl = pl.ds(q * 16, 16)
                            sh = pl.ds(HID + q * 16, 16)
                            ve = (ebuf[b][i, sl] + sbuf[b][2 * i, sl]
                                  + rbuf[b][2 * i, sl])
                            vo = (ebuf[b][i, sh] + sbuf[b][2 * i + 1, sl]
                                  + rbuf[b][2 * i + 1, sl])
                            ebuf[b][i, sl] = ve
                            ebuf[b][i, sh] = vo
                            sbuf[b][2 * i, sl] = ve
                            sbuf[b][2 * i + 1, sl] = vo
                        return 0
                    lax.fori_loop(0, _CROWS, addrow, 0)

                    pltpu.sync_copy(sbuf[b], agg_sh.at[idxr6[u]], add=True)
                    pltpu.sync_copy(ebuf[b],
                                    msg_hbm.at[pl.ds(t * _CROWS, _CROWS)])

                b2 = (b + 2) % 3
                u2 = (u + 2) % 6
                t2 = wid + _NW * (j + 2)
                t3 = wid + _NW * (j + 3)

                @pl.when(t2 < _NCHUNKS)
                def _():
                    drain_idx()
                    issue_inputs(j + 2, b2, u2)

                @pl.when(t3 < _NCHUNKS)
                def _():
                    idx_async(j + 3, (u + 3) % 6)
            return 0
        lax.fori_loop(0, (_NITER + 5) // 6, outer, 0)

        plsc.subcore_barrier()

        @pl.when(sid < 15)
        def _():
            for off, nr in ((0, 160), (160, 160), (320, 160), (480, 144)):
                pltpu.sync_copy(agg_sh.at[pl.ds(sbase + off, nr)],
                                stage.at[pl.ds(0, nr)])
                pltpu.sync_copy(stage.at[pl.ds(0, nr)],
                                agg_hbm.at[cid, pl.ds(sbase + off, nr)])

        @pl.when(sid == 15)
        def _():
            for off in (0, 160, 320, 480):
                pltpu.sync_copy(agg_sh.at[pl.ds(15 * _STRIPE + off, 160)], stage)
                pltpu.sync_copy(stage,
                                agg_hbm.at[cid, pl.ds(15 * _STRIPE + off, 160)])

    return body(E2, edge_index, S, R)


def kernel(x, edge_attr, edge_index, params):
    node_h = _mlp_rows(x, params['nb_encoder'], 2000)
    ea2 = edge_attr.reshape(N_PAIR, 32)
    h_prev = _mlp_rows(ea2, _blockdiag_params(params['eb_encoder']), 4000,
                       d_out=2 * HID)

    msg2 = None
    for k, blk in enumerate(params['blocks']):
        S, R = _sr_mlps(node_h, blk['sender'], blk['receiver'], 2000)
        ebd = _blockdiag_params(blk['edge'])
        if k == 0:
            E2 = _mlp_rows(h_prev, ebd, 4000, d_out=2 * HID)
        else:
            E2, h_prev = _fused_add_mlp(msg2, h_prev, ebd, 4000)
        msg2, agg_parts = _sc_edge_pass(E2, edge_index, S, R)
        node_h = _node_block2(node_h, agg_parts, blk['node'], 2000)

    return _mlp_rows(node_h, params['decoder'], 2000)
